# Initial kernel scaffold; baseline (speedup 1.0000x reference)
import jax
import jax.numpy as jnp
from jax.experimental import pallas as pl


def kernel(x, node_attrs, edge_index, atomic_numbers):
    raise NotImplementedError("write your pallas kernel here")


# speedup vs baseline: 230.8189x; 230.8189x over previous
import numpy as np
import jax
import jax.numpy as jnp
from jax import lax
from jax.experimental import pallas as pl
from jax.experimental.pallas import tpu as pltpu
from jax.experimental.pallas import tpu_sc as plsc

_RADII = np.array([
    0.2, 0.31, 0.28, 1.28, 0.96, 0.84, 0.76, 0.71, 0.66, 0.57,
    0.58, 1.66, 1.41, 1.21, 1.11, 1.07, 1.05, 1.02, 1.06, 2.03,
    1.76, 1.70, 1.60, 1.53, 1.39, 1.39, 1.32, 1.26, 1.24, 1.32,
    1.22, 1.22, 1.20, 1.19, 1.20, 1.20, 1.16, 2.20, 1.95, 1.90,
    1.75, 1.64, 1.54, 1.47, 1.46, 1.42, 1.39, 1.45, 1.44, 1.42,
    1.39, 1.39, 1.38, 1.39, 1.40, 2.44, 2.15, 2.07, 2.04, 2.03,
    2.01, 1.99, 1.98, 1.98, 1.96, 1.94, 1.92, 1.92, 1.89, 1.90,
    1.87, 1.87, 1.75, 1.70, 1.62, 1.51, 1.44, 1.41, 1.36, 1.36,
    1.32, 1.45, 1.46, 1.48, 1.40, 1.50, 1.50, 2.60, 2.21, 2.15,
    2.06, 2.00, 1.96, 1.90, 1.87, 1.80, 1.69, 0.2, 0.2, 0.2,
    0.2, 0.2, 0.2, 0.2, 0.2, 0.2, 0.2, 0.2, 0.2, 0.2,
    0.2, 0.2, 0.2, 0.2, 0.2, 0.2, 0.2, 0.2, 0.2], dtype=np.float32)

_Q = np.float32(0.9183)
_P = np.float32(4.5791)
_A = np.float32(1.0805)
_LN2 = np.float32(0.6931471805599453)

_LN1P = [np.float32(c) for c in (
    2.4139036325365737e-09, 0.9999996692323803, -0.4999887596399388,
    0.33316691900963746, -0.2486582066539632, 0.193376371107157,
    -0.1451764592055529, 0.0947037960697186, -0.04713346543609073,
    0.015145372355873089, -0.00228806042676267)]

_N_NODES = 100000
_N_ELEM = 10
_E = 6400000
_N_PAD = 100096
_NC, _NS, _L = 2, 16, 16
_NW = _NC * _NS
_NPT = _N_PAD // _NS
_A_G = 23
_A_CH = 17
_A_NODES = _A_G * _L
_EPT = _E // _NW
_B_CH = 50
_B_N = _EPT // _B_CH
_B_G = _B_N // _L


def _ln(v):
    bits = plsc.bitcast(v, jnp.int32)
    ex = (bits >> 23) - 127
    m = plsc.bitcast((bits & 0x007FFFFF) | 0x3F800000, jnp.float32)
    t = m - np.float32(1.0)
    p = t * np.float32(0.0) + _LN1P[10]
    for k in range(9, -1, -1):
        p = p * t + _LN1P[k]
    return ex.astype(jnp.float32) * _LN2 + p


def _sc_body(x_hbm, attrs_hbm, snd_hbm, rcv_hbm, radii_hbm, anum_hbm,
             out_hbm,
             radii_v, anum_v, radelem_v, attrs_v, radch_v, table_sh, table_v,
             s_v, r_v, xx_v, o_v):
    cid = lax.axis_index("c")
    sid = lax.axis_index("s")
    iota10 = lax.iota(jnp.int32, _L) * _N_ELEM

    pltpu.sync_copy(radii_hbm, radii_v)
    pltpu.sync_copy(anum_hbm, anum_v)
    radelem_v[...] = plsc.load_gather(radii_v, [anum_v[...]])

    def a_chunk(c, carry):
        off = sid * (_NPT * _N_ELEM) + c * (_A_NODES * _N_ELEM)
        pltpu.sync_copy(attrs_hbm.at[pl.ds(off, _A_NODES * _N_ELEM)], attrs_v)

        def a_group(g, carry2):
            base = g * (_L * _N_ELEM) + iota10
            mx = plsc.load_gather(attrs_v, [base])
            am = jnp.zeros((_L,), jnp.int32)
            for j in range(1, _N_ELEM):
                val = plsc.load_gather(attrs_v, [base + j])
                upd = val > mx
                mx = jnp.where(upd, val, mx)
                am = jnp.where(upd, j, am)
            radch_v[pl.ds(g * _L, _L)] = plsc.load_gather(radelem_v, [am])
            return carry2

        lax.fori_loop(0, _A_G, a_group, 0)
        pltpu.sync_copy(radch_v,
                        table_sh.at[pl.ds(sid * _NPT + c * _A_NODES, _A_NODES)])
        return carry

    lax.fori_loop(0, _A_CH, a_chunk, 0)
    plsc.subcore_barrier()
    pltpu.sync_copy(table_sh, table_v)

    ebase = (cid * _NS + sid) * _EPT

    def b_chunk(c, carry):
        off = ebase + c * _B_N
        pltpu.sync_copy(snd_hbm.at[pl.ds(off, _B_N)], s_v)
        pltpu.sync_copy(rcv_hbm.at[pl.ds(off, _B_N)], r_v)
        pltpu.sync_copy(x_hbm.at[pl.ds(off, _B_N)], xx_v)

        def b_group(g, carry2):
            sl = pl.ds(g * _L, _L)
            ru = plsc.load_gather(table_v, [s_v[sl]])
            rw = plsc.load_gather(table_v, [r_v[sl]])
            r0 = np.float32(0.5) * (ru + rw)
            u = _ln(xx_v[sl] / r0)
            u = jnp.minimum(jnp.maximum(u, np.float32(-24.0)), np.float32(24.0))
            e1 = jnp.exp(_Q * u)
            e2 = jnp.exp((_Q - _P) * u)
            d = np.float32(1.0) + e2
            o_v[sl] = d / (d + _A * e1)
            return carry2

        lax.fori_loop(0, _B_G, b_group, 0)
        pltpu.sync_copy(o_v, out_hbm.at[pl.ds(off, _B_N)])
        return carry

    lax.fori_loop(0, _B_CH, b_chunk, 0)


def kernel(x, node_attrs, edge_index, atomic_numbers):
    x_flat = x.reshape(_E)
    attrs_flat = jnp.pad(node_attrs, ((0, _N_PAD - _N_NODES), (0, 0))
                         ).reshape(_N_PAD * _N_ELEM)
    snd = edge_index[0]
    rcv = edge_index[1]
    radii128 = jnp.pad(jnp.asarray(_RADII), (0, 128 - _RADII.shape[0]))
    anum16 = jnp.pad(atomic_numbers, (0, 16 - _N_ELEM))

    mesh = plsc.VectorSubcoreMesh(core_axis_name="c", subcore_axis_name="s")
    f = pl.kernel(
        _sc_body,
        out_type=jax.ShapeDtypeStruct((_E,), jnp.float32),
        mesh=mesh,
        compiler_params=pltpu.CompilerParams(needs_layout_passes=False),
        scratch_types=[
            pltpu.VMEM((128,), jnp.float32),
            pltpu.VMEM((16,), jnp.int32),
            pltpu.VMEM((16,), jnp.float32),
            pltpu.VMEM((_A_NODES * _N_ELEM,), jnp.float32),
            pltpu.VMEM((_A_NODES,), jnp.float32),
            pltpu.VMEM_SHARED((_N_PAD,), jnp.float32),
            pltpu.VMEM((_N_PAD,), jnp.float32),
            pltpu.VMEM((_B_N,), jnp.int32),
            pltpu.VMEM((_B_N,), jnp.int32),
            pltpu.VMEM((_B_N,), jnp.float32),
            pltpu.VMEM((_B_N,), jnp.float32),
        ],
    )
    out = f(x_flat, attrs_flat, snd, rcv, radii128, anum16)
    return out.reshape(_E, 1)


# speedup vs baseline: 690.6057x; 2.9920x over previous
import numpy as np
import jax
import jax.numpy as jnp
from jax import lax
from jax.experimental import pallas as pl
from jax.experimental.pallas import tpu as pltpu
from jax.experimental.pallas import tpu_sc as plsc

_RADII = np.array([
    0.2, 0.31, 0.28, 1.28, 0.96, 0.84, 0.76, 0.71, 0.66, 0.57,
    0.58, 1.66, 1.41, 1.21, 1.11, 1.07, 1.05, 1.02, 1.06, 2.03,
    1.76, 1.70, 1.60, 1.53, 1.39, 1.39, 1.32, 1.26, 1.24, 1.32,
    1.22, 1.22, 1.20, 1.19, 1.20, 1.20, 1.16, 2.20, 1.95, 1.90,
    1.75, 1.64, 1.54, 1.47, 1.46, 1.42, 1.39, 1.45, 1.44, 1.42,
    1.39, 1.39, 1.38, 1.39, 1.40, 2.44, 2.15, 2.07, 2.04, 2.03,
    2.01, 1.99, 1.98, 1.98, 1.96, 1.94, 1.92, 1.92, 1.89, 1.90,
    1.87, 1.87, 1.75, 1.70, 1.62, 1.51, 1.44, 1.41, 1.36, 1.36,
    1.32, 1.45, 1.46, 1.48, 1.40, 1.50, 1.50, 2.60, 2.21, 2.15,
    2.06, 2.00, 1.96, 1.90, 1.87, 1.80, 1.69, 0.2, 0.2, 0.2,
    0.2, 0.2, 0.2, 0.2, 0.2, 0.2, 0.2, 0.2, 0.2, 0.2,
    0.2, 0.2, 0.2, 0.2, 0.2, 0.2, 0.2, 0.2, 0.2], dtype=np.float32)

_Q = np.float32(0.9183)
_P = np.float32(4.5791)
_A = np.float32(1.0805)
_LN2 = np.float32(0.6931471805599453)

_LN1P = [np.float32(c) for c in (
    2.4139036325365737e-09, 0.9999996692323803, -0.4999887596399388,
    0.33316691900963746, -0.2486582066539632, 0.193376371107157,
    -0.1451764592055529, 0.0947037960697186, -0.04713346543609073,
    0.015145372355873089, -0.00228806042676267)]

_N_NODES = 100000
_N_ELEM = 10
_E = 6400000
_N_PAD = 100096
_NC, _NS, _L = 2, 16, 16
_NW = _NC * _NS
_NPT = _N_PAD // _NS
_A_G = 23
_A_CH = 17
_A_NODES = _A_G * _L
_EPT = _E // _NW
_B_CH = 125
_B_N = _EPT // _B_CH
_B_G = _B_N // _L


def _ln(v):
    bits = plsc.bitcast(v, jnp.int32)
    ex = (bits >> 23) - 127
    m = plsc.bitcast((bits & 0x007FFFFF) | 0x3F800000, jnp.float32)
    t = m - np.float32(1.0)
    p = t * np.float32(0.0) + _LN1P[10]
    for k in range(9, -1, -1):
        p = p * t + _LN1P[k]
    return ex.astype(jnp.float32) * _LN2 + p


def _sc_body(x_hbm, attrs_hbm, snd_hbm, rcv_hbm, radii_hbm, anum_hbm,
             out_hbm,
             radii_v, anum_v, radelem_v, attrs_v, radch_v, table_sh, table_v,
             s_v, r_v, xx_v, o_v, sem_s, sem_r, sem_x, sem_o):
    cid = lax.axis_index("c")
    sid = lax.axis_index("s")
    iota10 = lax.iota(jnp.int32, _L) * _N_ELEM

    pltpu.sync_copy(radii_hbm, radii_v)
    pltpu.sync_copy(anum_hbm, anum_v)
    radelem_v[...] = plsc.load_gather(radii_v, [anum_v[...]])

    def a_chunk(c, carry):
        off = sid * (_NPT * _N_ELEM) + c * (_A_NODES * _N_ELEM)
        pltpu.sync_copy(attrs_hbm.at[pl.ds(off, _A_NODES * _N_ELEM)], attrs_v)

        @plsc.parallel_loop(0, _A_G, unroll=2)
        def a_group(g):
            base = g * (_L * _N_ELEM) + iota10
            mx = plsc.load_gather(attrs_v, [base])
            am = jnp.zeros((_L,), jnp.int32)
            for j in range(1, _N_ELEM):
                val = plsc.load_gather(attrs_v, [base + j])
                upd = val > mx
                mx = jnp.where(upd, val, mx)
                am = jnp.where(upd, j, am)
            radch_v[pl.ds(g * _L, _L)] = plsc.load_gather(radelem_v, [am])

        pltpu.sync_copy(radch_v,
                        table_sh.at[pl.ds(sid * _NPT + c * _A_NODES, _A_NODES)])
        return carry

    lax.fori_loop(0, _A_CH, a_chunk, 0)
    plsc.subcore_barrier()
    pltpu.sync_copy(table_sh, table_v)

    ebase = (cid * _NS + sid) * _EPT

    def in_descs(c):
        b = lax.rem(c, 2)
        off = ebase + c * _B_N
        dst = pl.ds(b * _B_N, _B_N)
        return (
            (snd_hbm.at[pl.ds(off, _B_N)], s_v.at[dst], sem_s.at[b]),
            (rcv_hbm.at[pl.ds(off, _B_N)], r_v.at[dst], sem_r.at[b]),
            (x_hbm.at[pl.ds(off, _B_N)], xx_v.at[dst], sem_x.at[b]),
        )

    def out_desc(c):
        b = lax.rem(c, 2)
        off = ebase + c * _B_N
        return (o_v.at[pl.ds(b * _B_N, _B_N)], out_hbm.at[pl.ds(off, _B_N)],
                sem_o.at[b])

    def start_in(c):
        for src, dst, sem in in_descs(c):
            pltpu.async_copy(src, dst, sem)

    start_in(0)

    def b_chunk(c, carry):
        @pl.when(c + 1 < _B_CH)
        def _prefetch():
            start_in(c + 1)

        for src, dst, sem in in_descs(c):
            pltpu.make_async_copy(src, dst, sem).wait()

        @pl.when(c >= 2)
        def _drain_out():
            src, dst, sem = out_desc(c - 2)
            pltpu.make_async_copy(src, dst, sem).wait()

        b = lax.rem(c, 2)
        bo = b * _B_N

        @plsc.parallel_loop(0, _B_G, unroll=8)
        def b_group(g):
            sl = pl.ds(bo + g * _L, _L)
            ru = plsc.load_gather(table_v, [s_v[sl]])
            rw = plsc.load_gather(table_v, [r_v[sl]])
            r0 = np.float32(0.5) * (ru + rw)
            u = _ln(xx_v[sl] / r0)
            u = jnp.minimum(jnp.maximum(u, np.float32(-24.0)), np.float32(24.0))
            e1 = jnp.exp(_Q * u)
            e2 = jnp.exp((_Q - _P) * u)
            d = np.float32(1.0) + e2
            o_v[sl] = d / (d + _A * e1)

        src, dst, sem = out_desc(c)
        pltpu.async_copy(src, dst, sem)
        return carry

    lax.fori_loop(0, _B_CH, b_chunk, 0)
    for c in (_B_CH - 2, _B_CH - 1):
        src, dst, sem = out_desc(c)
        pltpu.make_async_copy(src, dst, sem).wait()


def kernel(x, node_attrs, edge_index, atomic_numbers):
    x_flat = x.reshape(_E)
    attrs_flat = jnp.pad(node_attrs, ((0, _N_PAD - _N_NODES), (0, 0))
                         ).reshape(_N_PAD * _N_ELEM)
    snd = edge_index[0]
    rcv = edge_index[1]
    radii128 = jnp.pad(jnp.asarray(_RADII), (0, 128 - _RADII.shape[0]))
    anum16 = jnp.pad(atomic_numbers, (0, 16 - _N_ELEM))

    mesh = plsc.VectorSubcoreMesh(core_axis_name="c", subcore_axis_name="s")
    f = pl.kernel(
        _sc_body,
        out_type=jax.ShapeDtypeStruct((_E,), jnp.float32),
        mesh=mesh,
        compiler_params=pltpu.CompilerParams(needs_layout_passes=False),
        scratch_types=[
            pltpu.VMEM((128,), jnp.float32),
            pltpu.VMEM((16,), jnp.int32),
            pltpu.VMEM((16,), jnp.float32),
            pltpu.VMEM((_A_NODES * _N_ELEM,), jnp.float32),
            pltpu.VMEM((_A_NODES,), jnp.float32),
            pltpu.VMEM_SHARED((_N_PAD,), jnp.float32),
            pltpu.VMEM((_N_PAD,), jnp.float32),
            pltpu.VMEM((2 * _B_N,), jnp.int32),
            pltpu.VMEM((2 * _B_N,), jnp.int32),
            pltpu.VMEM((2 * _B_N,), jnp.float32),
            pltpu.VMEM((2 * _B_N,), jnp.float32),
            pltpu.SemaphoreType.DMA((2,)),
            pltpu.SemaphoreType.DMA((2,)),
            pltpu.SemaphoreType.DMA((2,)),
            pltpu.SemaphoreType.DMA((2,)),
        ],
    )
    out = f(x_flat, attrs_flat, snd, rcv, radii128, anum16)
    return out.reshape(_E, 1)


# speedup vs baseline: 826.4429x; 1.1967x over previous
import numpy as np
import jax
import jax.numpy as jnp
from jax import lax
from jax.experimental import pallas as pl
from jax.experimental.pallas import tpu as pltpu
from jax.experimental.pallas import tpu_sc as plsc

_RADII = np.array([
    0.2, 0.31, 0.28, 1.28, 0.96, 0.84, 0.76, 0.71, 0.66, 0.57,
    0.58, 1.66, 1.41, 1.21, 1.11, 1.07, 1.05, 1.02, 1.06, 2.03,
    1.76, 1.70, 1.60, 1.53, 1.39, 1.39, 1.32, 1.26, 1.24, 1.32,
    1.22, 1.22, 1.20, 1.19, 1.20, 1.20, 1.16, 2.20, 1.95, 1.90,
    1.75, 1.64, 1.54, 1.47, 1.46, 1.42, 1.39, 1.45, 1.44, 1.42,
    1.39, 1.39, 1.38, 1.39, 1.40, 2.44, 2.15, 2.07, 2.04, 2.03,
    2.01, 1.99, 1.98, 1.98, 1.96, 1.94, 1.92, 1.92, 1.89, 1.90,
    1.87, 1.87, 1.75, 1.70, 1.62, 1.51, 1.44, 1.41, 1.36, 1.36,
    1.32, 1.45, 1.46, 1.48, 1.40, 1.50, 1.50, 2.60, 2.21, 2.15,
    2.06, 2.00, 1.96, 1.90, 1.87, 1.80, 1.69, 0.2, 0.2, 0.2,
    0.2, 0.2, 0.2, 0.2, 0.2, 0.2, 0.2, 0.2, 0.2, 0.2,
    0.2, 0.2, 0.2, 0.2, 0.2, 0.2, 0.2, 0.2, 0.2], dtype=np.float32)

_Q = np.float32(0.9183)
_P = np.float32(4.5791)
_A = np.float32(1.0805)
_LN2 = np.float32(0.6931471805599453)

_LN1P = [np.float32(c) for c in (
    2.4139036325365737e-09, 0.9999996692323803, -0.4999887596399388,
    0.33316691900963746, -0.2486582066539632, 0.193376371107157,
    -0.1451764592055529, 0.0947037960697186, -0.04713346543609073,
    0.015145372355873089, -0.00228806042676267)]

_N_NODES = 100000
_N_ELEM = 10
_E = 6400000
_NC, _NS, _L = 2, 16, 16
_NW = _NC * _NS
_A_NODES = 400
_A_G = _A_NODES // _L
_A_CH = _N_NODES // _A_NODES
_A_IT = 16
_EPT = _E // _NW
_B_CH = 125
_B_N = _EPT // _B_CH
_B_G = _B_N // _L


def _ln(v):
    bits = plsc.bitcast(v, jnp.int32)
    ex = (bits >> 23) - 127
    m = plsc.bitcast((bits & 0x007FFFFF) | 0x3F800000, jnp.float32)
    t = m - np.float32(1.0)
    p = t * np.float32(0.0) + _LN1P[10]
    for k in range(9, -1, -1):
        p = p * t + _LN1P[k]
    return ex.astype(jnp.float32) * _LN2 + p


def _sc_body(x_hbm, attrs_hbm, edge_hbm, radii_hbm, anum_hbm,
             out_hbm,
             radii_v, anum_v, radelem_v, attrs_v, radch_v, table_sh, table_v,
             s_v, r_v, xx_v, o_v, sem_s, sem_r, sem_x, sem_o):
    cid = lax.axis_index("c")
    sid = lax.axis_index("s")
    iota = lax.iota(jnp.int32, _L)

    pltpu.sync_copy(radii_hbm, radii_v)
    pltpu.sync_copy(anum_hbm, anum_v)
    radelem_v[...] = plsc.load_gather(radii_v, [anum_v[...]])

    def a_chunk(i, carry):
        c = i * _NS + sid

        @pl.when(c < _A_CH)
        def _do():
            pltpu.sync_copy(
                attrs_hbm.at[pl.ds(c * _A_NODES * _N_ELEM, _A_NODES * _N_ELEM)],
                attrs_v)

            @plsc.parallel_loop(0, _A_G, unroll=2)
            def a_group(g):
                rows = (g * _L + iota) * _N_ELEM
                mx = plsc.load_gather(attrs_v, [rows])
                am = jnp.zeros((_L,), jnp.int32)
                for j in range(1, _N_ELEM):
                    val = plsc.load_gather(attrs_v, [rows + j])
                    upd = val > mx
                    mx = jnp.where(upd, val, mx)
                    am = jnp.where(upd, j, am)
                radch_v[pl.ds(g * _L, _L)] = plsc.load_gather(radelem_v, [am])

            pltpu.sync_copy(radch_v, table_sh.at[pl.ds(c * _A_NODES, _A_NODES)])

        return carry

    lax.fori_loop(0, _A_IT, a_chunk, 0)
    plsc.subcore_barrier()
    pltpu.sync_copy(table_sh, table_v)

    ebase = (cid * _NS + sid) * _EPT

    def in_descs(c):
        b = lax.rem(c, 2)
        off = ebase + c * _B_N
        dst = pl.ds(b * _B_N, _B_N)
        return (
            (edge_hbm.at[pl.ds(off, _B_N)], s_v.at[dst], sem_s.at[b]),
            (edge_hbm.at[pl.ds(_E + off, _B_N)], r_v.at[dst], sem_r.at[b]),
            (x_hbm.at[pl.ds(off, _B_N)], xx_v.at[dst], sem_x.at[b]),
        )

    def out_desc(c):
        b = lax.rem(c, 2)
        off = ebase + c * _B_N
        return (o_v.at[pl.ds(b * _B_N, _B_N)], out_hbm.at[pl.ds(off, _B_N)],
                sem_o.at[b])

    def start_in(c):
        for src, dst, sem in in_descs(c):
            pltpu.async_copy(src, dst, sem)

    start_in(0)

    def b_chunk(c, carry):
        @pl.when(c + 1 < _B_CH)
        def _prefetch():
            start_in(c + 1)

        for src, dst, sem in in_descs(c):
            pltpu.make_async_copy(src, dst, sem).wait()

        @pl.when(c >= 2)
        def _drain_out():
            src, dst, sem = out_desc(c - 2)
            pltpu.make_async_copy(src, dst, sem).wait()

        b = lax.rem(c, 2)
        bo = b * _B_N

        @plsc.parallel_loop(0, _B_G, unroll=8)
        def b_group(g):
            sl = pl.ds(bo + g * _L, _L)
            ru = plsc.load_gather(table_v, [s_v[sl]])
            rw = plsc.load_gather(table_v, [r_v[sl]])
            r0 = np.float32(0.5) * (ru + rw)
            u = _ln(xx_v[sl] / r0)
            u = jnp.minimum(jnp.maximum(u, np.float32(-24.0)), np.float32(24.0))
            e1 = jnp.exp(_Q * u)
            e2 = jnp.exp((_Q - _P) * u)
            d = np.float32(1.0) + e2
            o_v[sl] = d / (d + _A * e1)

        src, dst, sem = out_desc(c)
        pltpu.async_copy(src, dst, sem)
        return carry

    lax.fori_loop(0, _B_CH, b_chunk, 0)
    for c in (_B_CH - 2, _B_CH - 1):
        src, dst, sem = out_desc(c)
        pltpu.make_async_copy(src, dst, sem).wait()


def kernel(x, node_attrs, edge_index, atomic_numbers):
    radii128 = jnp.pad(jnp.asarray(_RADII), (0, 128 - _RADII.shape[0]))
    anum16 = jnp.pad(atomic_numbers, (0, 16 - _N_ELEM))
    x_flat = x.reshape(_E)
    attrs_flat = node_attrs.reshape(_N_NODES * _N_ELEM)
    edge_flat = edge_index.reshape(2 * _E)

    mesh = plsc.VectorSubcoreMesh(core_axis_name="c", subcore_axis_name="s")
    f = pl.kernel(
        _sc_body,
        out_type=jax.ShapeDtypeStruct((_E,), jnp.float32),
        mesh=mesh,
        compiler_params=pltpu.CompilerParams(needs_layout_passes=False),
        scratch_types=[
            pltpu.VMEM((128,), jnp.float32),
            pltpu.VMEM((16,), jnp.int32),
            pltpu.VMEM((16,), jnp.float32),
            pltpu.VMEM((_A_NODES * _N_ELEM,), jnp.float32),
            pltpu.VMEM((_A_NODES,), jnp.float32),
            pltpu.VMEM_SHARED((_N_NODES,), jnp.float32),
            pltpu.VMEM((_N_NODES,), jnp.float32),
            pltpu.VMEM((2 * _B_N,), jnp.int32),
            pltpu.VMEM((2 * _B_N,), jnp.int32),
            pltpu.VMEM((2 * _B_N,), jnp.float32),
            pltpu.VMEM((2 * _B_N,), jnp.float32),
            pltpu.SemaphoreType.DMA((2,)),
            pltpu.SemaphoreType.DMA((2,)),
            pltpu.SemaphoreType.DMA((2,)),
            pltpu.SemaphoreType.DMA((2,)),
        ],
    )
    return f(x_flat, attrs_flat, edge_flat, radii128, anum16).reshape(_E, 1)


# speedup vs baseline: 832.7034x; 1.0076x over previous
import numpy as np
import jax
import jax.numpy as jnp
from jax import lax
from jax.experimental import pallas as pl
from jax.experimental.pallas import tpu as pltpu
from jax.experimental.pallas import tpu_sc as plsc

_RADII = np.array([
    0.2, 0.31, 0.28, 1.28, 0.96, 0.84, 0.76, 0.71, 0.66, 0.57,
    0.58, 1.66, 1.41, 1.21, 1.11, 1.07, 1.05, 1.02, 1.06, 2.03,
    1.76, 1.70, 1.60, 1.53, 1.39, 1.39, 1.32, 1.26, 1.24, 1.32,
    1.22, 1.22, 1.20, 1.19, 1.20, 1.20, 1.16, 2.20, 1.95, 1.90,
    1.75, 1.64, 1.54, 1.47, 1.46, 1.42, 1.39, 1.45, 1.44, 1.42,
    1.39, 1.39, 1.38, 1.39, 1.40, 2.44, 2.15, 2.07, 2.04, 2.03,
    2.01, 1.99, 1.98, 1.98, 1.96, 1.94, 1.92, 1.92, 1.89, 1.90,
    1.87, 1.87, 1.75, 1.70, 1.62, 1.51, 1.44, 1.41, 1.36, 1.36,
    1.32, 1.45, 1.46, 1.48, 1.40, 1.50, 1.50, 2.60, 2.21, 2.15,
    2.06, 2.00, 1.96, 1.90, 1.87, 1.80, 1.69, 0.2, 0.2, 0.2,
    0.2, 0.2, 0.2, 0.2, 0.2, 0.2, 0.2, 0.2, 0.2, 0.2,
    0.2, 0.2, 0.2, 0.2, 0.2, 0.2, 0.2, 0.2, 0.2], dtype=np.float32)

_Q = np.float32(0.9183)
_P = np.float32(4.5791)
_A = np.float32(1.0805)
_LN2 = np.float32(0.6931471805599453)

_LN1P = [np.float32(c) for c in (
    2.4139036325365737e-09, 0.9999996692323803, -0.4999887596399388,
    0.33316691900963746, -0.2486582066539632, 0.193376371107157,
    -0.1451764592055529, 0.0947037960697186, -0.04713346543609073,
    0.015145372355873089, -0.00228806042676267)]

_N_NODES = 100000
_N_ELEM = 10
_E = 6400000
_NC, _NS, _L = 2, 16, 16
_NW = _NC * _NS
_A_NODES = 400
_A_G = _A_NODES // _L
_A_CH = _N_NODES // _A_NODES
_A_IT = 16
_EPT = _E // _NW
_B_CH = 100
_B_N = _EPT // _B_CH
_B_G = _B_N // _L


def _ln(v):
    bits = plsc.bitcast(v, jnp.int32)
    ex = (bits >> 23) - 127
    m = plsc.bitcast((bits & 0x007FFFFF) | 0x3F800000, jnp.float32)
    t = m - np.float32(1.0)
    p = t * np.float32(0.0) + _LN1P[10]
    for k in range(9, -1, -1):
        p = p * t + _LN1P[k]
    return ex.astype(jnp.float32) * _LN2 + p


def _sc_body(x_hbm, attrs_hbm, edge_hbm, radii_hbm, anum_hbm,
             out_hbm,
             radii_v, anum_v, radelem_v, attrs_v, radch_v, table_sh, table_v,
             s_v, r_v, xx_v, o_v, sem_s, sem_r, sem_x, sem_o):
    cid = lax.axis_index("c")
    sid = lax.axis_index("s")
    iota = lax.iota(jnp.int32, _L)

    pltpu.sync_copy(radii_hbm, radii_v)
    pltpu.sync_copy(anum_hbm, anum_v)
    radelem_v[...] = plsc.load_gather(radii_v, [anum_v[...]])

    def a_chunk(i, carry):
        c = i * _NS + sid

        @pl.when(c < _A_CH)
        def _do():
            pltpu.sync_copy(
                attrs_hbm.at[pl.ds(c * _A_NODES * _N_ELEM, _A_NODES * _N_ELEM)],
                attrs_v)

            @plsc.parallel_loop(0, _A_G, unroll=2)
            def a_group(g):
                rows = (g * _L + iota) * _N_ELEM
                mx = plsc.load_gather(attrs_v, [rows])
                am = jnp.zeros((_L,), jnp.int32)
                for j in range(1, _N_ELEM):
                    val = plsc.load_gather(attrs_v, [rows + j])
                    upd = val > mx
                    mx = jnp.where(upd, val, mx)
                    am = jnp.where(upd, j, am)
                radch_v[pl.ds(g * _L, _L)] = plsc.load_gather(radelem_v, [am])

            pltpu.sync_copy(radch_v, table_sh.at[pl.ds(c * _A_NODES, _A_NODES)])

        return carry

    lax.fori_loop(0, _A_IT, a_chunk, 0)
    plsc.subcore_barrier()
    pltpu.sync_copy(table_sh, table_v)

    ebase = (cid * _NS + sid) * _EPT

    def in_descs(c, b):
        off = ebase + c * _B_N
        dst = pl.ds(b * _B_N, _B_N)
        return (
            (edge_hbm.at[pl.ds(off, _B_N)], s_v.at[dst], sem_s.at[b]),
            (edge_hbm.at[pl.ds(_E + off, _B_N)], r_v.at[dst], sem_r.at[b]),
            (x_hbm.at[pl.ds(off, _B_N)], xx_v.at[dst], sem_x.at[b]),
        )

    def out_desc(c, b):
        off = ebase + c * _B_N
        return (o_v.at[pl.ds(b * _B_N, _B_N)], out_hbm.at[pl.ds(off, _B_N)],
                sem_o.at[b])

    def start_in(c, b):
        for src, dst, sem in in_descs(c, b):
            pltpu.async_copy(src, dst, sem)

    def compute(c, b):
        bo = b * _B_N

        @plsc.parallel_loop(0, _B_G, unroll=8)
        def b_group(g):
            sl = pl.ds(bo + g * _L, _L)
            ru = plsc.load_gather(table_v, [s_v[sl]])
            rw = plsc.load_gather(table_v, [r_v[sl]])
            r0 = np.float32(0.5) * (ru + rw)
            u = _ln(xx_v[sl] / r0)
            u = jnp.minimum(jnp.maximum(u, np.float32(-18.0)), np.float32(18.0))
            e1 = jnp.exp(_Q * u)
            e2 = jnp.exp((_Q - _P) * u)
            d = np.float32(1.0) + e2
            o_v[sl] = d / (d + _A * e1)

    start_in(0, 0)

    @pl.loop(0, _B_CH, step=2)
    def b_pair(c0):
        for b in (0, 1):
            c = c0 + b

            @pl.when(c + 1 < _B_CH)
            def _prefetch():
                start_in(c + 1, 1 - b)

            for src, dst, sem in in_descs(c, b):
                pltpu.make_async_copy(src, dst, sem).wait()

            @pl.when(c >= 2)
            def _drain_out():
                src, dst, sem = out_desc(c - 2, b)
                pltpu.make_async_copy(src, dst, sem).wait()

            compute(c, b)
            src, dst, sem = out_desc(c, b)
            pltpu.async_copy(src, dst, sem)

    for c, b in ((_B_CH - 2, 0), (_B_CH - 1, 1)):
        src, dst, sem = out_desc(c, b)
        pltpu.make_async_copy(src, dst, sem).wait()


def kernel(x, node_attrs, edge_index, atomic_numbers):
    radii128 = jnp.pad(jnp.asarray(_RADII), (0, 128 - _RADII.shape[0]))
    anum16 = jnp.pad(atomic_numbers, (0, 16 - _N_ELEM))
    x_flat = x.reshape(_E)
    attrs_flat = node_attrs.reshape(_N_NODES * _N_ELEM)
    edge_flat = edge_index.reshape(2 * _E)

    mesh = plsc.VectorSubcoreMesh(core_axis_name="c", subcore_axis_name="s")
    f = pl.kernel(
        _sc_body,
        out_type=jax.ShapeDtypeStruct((_E,), jnp.float32),
        mesh=mesh,
        compiler_params=pltpu.CompilerParams(needs_layout_passes=False),
        scratch_types=[
            pltpu.VMEM((128,), jnp.float32),
            pltpu.VMEM((16,), jnp.int32),
            pltpu.VMEM((16,), jnp.float32),
            pltpu.VMEM((_A_NODES * _N_ELEM,), jnp.float32),
            pltpu.VMEM((_A_NODES,), jnp.float32),
            pltpu.VMEM_SHARED((_N_NODES,), jnp.float32),
            pltpu.VMEM((_N_NODES,), jnp.float32),
            pltpu.VMEM((2 * _B_N,), jnp.int32),
            pltpu.VMEM((2 * _B_N,), jnp.int32),
            pltpu.VMEM((2 * _B_N,), jnp.float32),
            pltpu.VMEM((2 * _B_N,), jnp.float32),
            pltpu.SemaphoreType.DMA((2,)),
            pltpu.SemaphoreType.DMA((2,)),
            pltpu.SemaphoreType.DMA((2,)),
            pltpu.SemaphoreType.DMA((2,)),
        ],
    )
    return f(x_flat, attrs_flat, edge_flat, radii128, anum16).reshape(_E, 1)
